# baseline (speedup 1.0000x reference)
import functools

import jax
import jax.numpy as jnp
from jax import lax
from jax.experimental import pallas as pl
from jax.experimental.pallas import tpu as pltpu
from jax.experimental.pallas import tpu_sc as plsc

B, L = 1024, 200
S = 8
C = 16
TOK = B * L
ROWS = TOK // S
D_OUT = 128

_OFF_D = 0
_OFF_S0 = 8
_OFF_S1 = 8 + 128
_OFF_S2 = 8 + 256
_TCAT_ROWS = 8 + 3 * 128 + 8


_NW = 32
_CHUNK = 128
_TPW = TOK // _NW
_CPW = _TPW // _CHUNK
_PC = 10
_NPASS = _CPW // _PC


def _sc_body(tv, tcat, iv, id_, i0, i1, i2, out,
             bv, bd, b0, b1, b2, bufa, bufb,
             sva, svb, saa, sab, soa, sob):
    wid = lax.axis_index("s") * 2 + lax.axis_index("c")
    base = wid * _TPW
    idx_bufs = (bv, bd, b0, b1, b2)
    for ihbm, ivm in zip((iv, id_, i0, i1, i2), idx_bufs):
        pltpu.sync_copy(ihbm.at[pl.ds(base, _TPW)], ivm)

    banks = (bufa, bufb)
    semv = (sva, svb)
    sema = (saa, sab)
    semo = (soa, sob)

    def fire_val(p):
        descs = []
        for ci in range(_PC):
            off = (p * _PC + ci) * _CHUNK
            descs.append(pltpu.async_copy(
                tv.at[bv.at[pl.ds(off, _CHUNK)]],
                banks[p % 2].at[ci], semv[p % 2]))
        return descs

    def fire_add(p):
        descs = []
        for bidx in (bd, b0, b1, b2):
            for ci in range(_PC):
                off = (p * _PC + ci) * _CHUNK
                descs.append(pltpu.async_copy(
                    tcat.at[bidx.at[pl.ds(off, _CHUNK)]],
                    banks[p % 2].at[ci], sema[p % 2], add=True))
        return descs

    def fire_out(p):
        descs = []
        tok0 = base + p * _PC * _CHUNK
        for ci in range(_PC):
            descs.append(pltpu.async_copy(
                banks[p % 2].at[ci],
                out.at[pl.ds(tok0 + ci * _CHUNK, _CHUNK)], semo[p % 2]))
        return descs

    descs_v = fire_val(0)
    descs_o = ([], [])
    for p in range(_NPASS):
        for d in descs_v:
            d.wait()
        descs_a = fire_add(p)
        if p + 1 < _NPASS:
            for d in descs_o[(p + 1) % 2]:
                d.wait()
            descs_v = fire_val(p + 1)
        for d in descs_a:
            d.wait()
        if p % 2 == 0:
            descs_o = (fire_out(p), descs_o[1])
        else:
            descs_o = (descs_o[0], fire_out(p))
    for ds_ in descs_o:
        for d in ds_:
            d.wait()


def _sc_gather_sum(tv, tcat, iv, id_, i0, i1, i2):
    mesh = plsc.VectorSubcoreMesh(core_axis_name="c", subcore_axis_name="s")
    kern = functools.partial(
        pl.kernel,
        mesh=mesh,
        compiler_params=pltpu.CompilerParams(use_tc_tiling_on_sc=False),
        out_type=jax.ShapeDtypeStruct((TOK, C), jnp.float32),
        scratch_types=(
            [pltpu.VMEM((_TPW,), jnp.int32) for _ in range(5)]
            + [pltpu.VMEM((_PC, _CHUNK, C), jnp.float32) for _ in range(2)]
            + [pltpu.SemaphoreType.DMA for _ in range(6)]
        ),
    )(_sc_body)
    return kern(tv, tcat, iv, id_, i0, i1, i2)


_GRID = 32
_RB = ROWS // _GRID


def _tc_body(xv_ref, vid_ref, k_ref, wt_ref, b_ref, out_ref):
    hi = jax.lax.Precision.HIGHEST
    ind = (vid_ref[...] == 0).astype(jnp.float32)
    x = xv_ref[...] - jax.lax.dot(ind, k_ref[...], precision=hi)
    out_ref[...] = jax.lax.dot(x, wt_ref[...], precision=hi) + b_ref[...]


def _tc_fix_conv(xv, vid, k, wt, bias):
    def full(shape):
        return pl.BlockSpec(shape, lambda *_: tuple(0 for _ in shape))

    return pl.pallas_call(
        _tc_body,
        grid=(_GRID,),
        in_specs=[
            pl.BlockSpec((_RB, D_OUT), lambda i: (i, 0)),
            pl.BlockSpec((_RB, S), lambda i: (i, 0)),
            full((S, D_OUT)),
            full((S * C, D_OUT)),
            full((1, D_OUT)),
        ],
        out_specs=pl.BlockSpec((_RB, D_OUT), lambda i: (i, 0)),
        out_shape=jax.ShapeDtypeStruct((ROWS, D_OUT), jnp.float32),
    )(xv, vid, k, wt, bias)


def kernel(value, depth, position, tgt_value_emb, tgt_depth_emb,
           tgt_spatial_emb, conv_w, conv_b):
    value = value.astype(jnp.int32)
    depth = depth.astype(jnp.int32)
    position = position.astype(jnp.int32)

    d8 = jnp.zeros((8, C), jnp.float32).at[1:7].set(tgt_depth_emb[1:])
    se_z = tgt_spatial_emb.at[:, 0, :].set(0.0)
    tcat = jnp.concatenate(
        [d8, se_z[0], se_z[1], se_z[2], jnp.zeros((8, C), jnp.float32)],
        axis=0)

    iv = value.reshape(TOK)
    id_ = depth.reshape(TOK)
    i0 = position[:, :, 0].reshape(TOK) + _OFF_S0
    i1 = position[:, :, 1].reshape(TOK) + _OFF_S1
    i2 = position[:, :, 2].reshape(TOK) + _OFF_S2

    xsum = _sc_gather_sum(tgt_value_emb, tcat, iv, id_, i0, i1, i2)
    xv = xsum.reshape(ROWS, D_OUT)

    vid = value.reshape(ROWS, S)
    k = jnp.kron(jnp.eye(S, dtype=jnp.float32), tgt_value_emb[0][None, :])
    wt = conv_w.transpose(2, 1, 0).reshape(S * C, D_OUT)
    bias = conv_b.reshape(1, D_OUT)

    out = _tc_fix_conv(xv, vid, k, wt, bias)
    return out.reshape(B, L // S, D_OUT)


# speedup vs baseline: 2.6415x; 2.6415x over previous
import functools

import jax
import jax.numpy as jnp
from jax import lax
from jax.experimental import pallas as pl
from jax.experimental.pallas import tpu as pltpu
from jax.experimental.pallas import tpu_sc as plsc

B, L = 1024, 200
S = 8
C = 16
TOK = B * L
ROWS = TOK // S
D_OUT = 128

_OFF_PAIR1 = 7 * 128


_NW = 32
_CHUNK = 128
_TPW = TOK // _NW
_CPW = _TPW // _CHUNK
_PC = 10
_NPASS = _CPW // _PC


def _sc_body(tv, tcat, iv, j0, j1, out,
             bv, b0, b1, bufa, bufb,
             sva, svb, saa, sab, soa, sob):
    wid = lax.axis_index("s") * 2 + lax.axis_index("c")
    base = wid * _TPW
    for ihbm, ivm in zip((iv, j0, j1), (bv, b0, b1)):
        pltpu.sync_copy(ihbm.at[pl.ds(base, _TPW)], ivm)

    banks = (bufa, bufb)
    semv = (sva, svb)
    sema = (saa, sab)
    semo = (soa, sob)

    def fire_val(p):
        descs = []
        for ci in range(_PC):
            off = (p * _PC + ci) * _CHUNK
            descs.append(pltpu.async_copy(
                tv.at[bv.at[pl.ds(off, _CHUNK)]],
                banks[p % 2].at[ci], semv[p % 2]))
        return descs

    def fire_add(p):
        descs = []
        for bidx in (b0, b1):
            for ci in range(_PC):
                off = (p * _PC + ci) * _CHUNK
                descs.append(pltpu.async_copy(
                    tcat.at[bidx.at[pl.ds(off, _CHUNK)]],
                    banks[p % 2].at[ci], sema[p % 2], add=True))
        return descs

    def fire_out(p):
        descs = []
        tok0 = base + p * _PC * _CHUNK
        for ci in range(_PC):
            descs.append(pltpu.async_copy(
                banks[p % 2].at[ci],
                out.at[pl.ds(tok0 + ci * _CHUNK, _CHUNK)], semo[p % 2]))
        return descs

    descs_v = fire_val(0)
    descs_o = ([], [])
    for p in range(_NPASS):
        for d in descs_v:
            d.wait()
        descs_a = fire_add(p)
        if p + 1 < _NPASS:
            for d in descs_o[(p + 1) % 2]:
                d.wait()
            descs_v = fire_val(p + 1)
        for d in descs_a:
            d.wait()
        if p % 2 == 0:
            descs_o = (fire_out(p), descs_o[1])
        else:
            descs_o = (descs_o[0], fire_out(p))
    for ds_ in descs_o:
        for d in ds_:
            d.wait()


def _sc_gather_sum(tv, tcat, iv, j0, j1):
    mesh = plsc.VectorSubcoreMesh(core_axis_name="c", subcore_axis_name="s")
    kern = functools.partial(
        pl.kernel,
        mesh=mesh,
        compiler_params=pltpu.CompilerParams(use_tc_tiling_on_sc=False),
        out_type=jax.ShapeDtypeStruct((TOK, C), jnp.float32),
        scratch_types=(
            [pltpu.VMEM((_TPW,), jnp.int32) for _ in range(3)]
            + [pltpu.VMEM((_PC, _CHUNK, C), jnp.float32) for _ in range(2)]
            + [pltpu.SemaphoreType.DMA for _ in range(6)]
        ),
    )(_sc_body)
    return kern(tv, tcat, iv, j0, j1)


_GRID = 32
_RB = ROWS // _GRID


def _tc_body(xv_ref, vid_ref, k_ref, wt_ref, b_ref, out_ref):
    hi = jax.lax.Precision.HIGHEST
    ind = (vid_ref[...] == 0).astype(jnp.float32)
    x = xv_ref[...] - jax.lax.dot(ind, k_ref[...], precision=hi)
    out_ref[...] = jax.lax.dot(x, wt_ref[...], precision=hi) + b_ref[...]


def _tc_fix_conv(xv, vid, k, wt, bias):
    def full(shape):
        return pl.BlockSpec(shape, lambda *_: tuple(0 for _ in shape))

    return pl.pallas_call(
        _tc_body,
        grid=(_GRID,),
        in_specs=[
            pl.BlockSpec((_RB, D_OUT), lambda i: (i, 0)),
            pl.BlockSpec((_RB, S), lambda i: (i, 0)),
            full((S, D_OUT)),
            full((S * C, D_OUT)),
            full((1, D_OUT)),
        ],
        out_specs=pl.BlockSpec((_RB, D_OUT), lambda i: (i, 0)),
        out_shape=jax.ShapeDtypeStruct((ROWS, D_OUT), jnp.float32),
    )(xv, vid, k, wt, bias)


def kernel(value, depth, position, tgt_value_emb, tgt_depth_emb,
           tgt_spatial_emb, conv_w, conv_b):
    value = value.astype(jnp.int32)
    depth = depth.astype(jnp.int32)
    position = position.astype(jnp.int32)

    de_z = tgt_depth_emb.at[0].set(0.0)
    se_z = tgt_spatial_emb.at[:, 0, :].set(0.0)
    t_d_s0 = (de_z[:, None, :] + se_z[0][None, :, :]).reshape(7 * 128, C)
    t_s1_s2 = (se_z[1][:, None, :] + se_z[2][None, :, :]).reshape(128 * 128, C)
    tcat = jnp.concatenate(
        [t_d_s0, t_s1_s2, jnp.zeros((8, C), jnp.float32)], axis=0)

    iv = value.reshape(TOK)
    j0 = (depth * 128 + position[:, :, 0]).reshape(TOK)
    j1 = (position[:, :, 1] * 128 + position[:, :, 2]).reshape(TOK) + _OFF_PAIR1

    xsum = _sc_gather_sum(tgt_value_emb, tcat, iv, j0, j1)
    xv = xsum.reshape(ROWS, D_OUT)

    vid = value.reshape(ROWS, S)
    k = jnp.kron(jnp.eye(S, dtype=jnp.float32), tgt_value_emb[0][None, :])
    wt = conv_w.transpose(2, 1, 0).reshape(S * C, D_OUT)
    bias = conv_b.reshape(1, D_OUT)

    out = _tc_fix_conv(xv, vid, k, wt, bias)
    return out.reshape(B, L // S, D_OUT)
